# Initial kernel scaffold; baseline (speedup 1.0000x reference)
import jax
import jax.numpy as jnp
from jax.experimental import pallas as pl


def kernel(student_feats, teacher_feats, mask, original_x, center):
    raise NotImplementedError("write your pallas kernel here")


# speedup vs baseline: 1.8591x; 1.8591x over previous
import functools

import jax
import jax.numpy as jnp
from jax import lax
from jax.experimental import pallas as pl
from jax.experimental.pallas import tpu as pltpu

_COV_W = 0.001
_PB = 2048


def _loss_kernel(s_ref, t_ref, orig_ref, mask_ref, center_ref, out_ref,
                 gram_acc, sums_acc, img_acc, *, nb, b_total):
    b = pl.program_id(0)
    j = pl.program_id(1)
    D = s_ref.shape[1]

    @pl.when(jnp.logical_and(b == 0, j == 0))
    def _init():
        gram_acc[...] = jnp.zeros_like(gram_acc)
        sums_acc[...] = jnp.zeros_like(sums_acc)
        img_acc[...] = jnp.zeros_like(img_acc)
        out_ref[...] = jnp.zeros_like(out_ref)

    x = s_ref[0]
    y = t_ref[0] - center_ref[...]

    inv_s = 1.0 / jnp.maximum(jnp.sqrt(jnp.sum(x * x, axis=0, keepdims=True)), 1e-12)
    sn = x * inv_s
    inv_t = 1.0 / jnp.maximum(jnp.sqrt(jnp.sum(y * y, axis=0, keepdims=True)), 1e-12)
    tn = y * inv_t

    et = jnp.exp(tn)
    zt = jnp.sum(et, axis=0, keepdims=True)
    log_zt = jnp.log(zt)
    es = jnp.exp(sn)
    log_zs = jnp.log(jnp.sum(es, axis=0, keepdims=True))

    inv_zt = 1.0 / zt
    dot_ts = jnp.sum(et * sn, axis=0, keepdims=True) * inv_zt
    dot_tt = jnp.sum(et * tn, axis=0, keepdims=True) * inv_zt

    fm = jnp.logical_and(orig_ref[0] != 0.0, mask_ref[0] == 0.0)
    fm = fm.astype(jnp.float32)

    loss_px = (log_zs - dot_ts) * fm
    tent_px = (log_zt - dot_tt) * fm

    blk_loss = jnp.sum(loss_px)
    blk_tent = jnp.sum(tent_px)
    blk_cnt = jnp.sum(fm)

    lane8 = lax.broadcasted_iota(jnp.int32, (8, 128), 1)
    sub8 = lax.broadcasted_iota(jnp.int32, (8, 128), 0)
    row = jnp.where(lane8 == 0, blk_loss,
                    jnp.where(lane8 == 1, blk_tent,
                              jnp.where(lane8 == 2, blk_cnt, 0.0)))
    img_acc[...] += jnp.where(sub8 == b, row, 0.0)

    sm = sn * fm
    gram_acc[...] += lax.dot_general(
        sm, sn, (((1,), (1,)), ((), ())),
        preferred_element_type=jnp.float32,
        precision=lax.Precision.HIGHEST)
    laneD = lax.broadcasted_iota(jnp.int32, (D, 128), 1)
    sums_acc[...] += jnp.where(laneD == 0,
                               jnp.sum(sm, axis=1, keepdims=True), 0.0)

    @pl.when(jnp.logical_and(b == b_total - 1, j == nb - 1))
    def _epilogue():
        imgs = img_acc[...]
        loss_sums = imgs[:, 0:1]
        tent_sums = imgs[:, 1:2]
        counts = imgs[:, 2:3]
        nf = jnp.sum(counts)

        g = gram_acc[...]
        m = sums_acc[:, 0:1] / nf
        gm = lax.dot_general(g, m, (((1,), (0,)), ((), ())),
                             preferred_element_type=jnp.float32,
                             precision=lax.Precision.HIGHEST)
        ri = lax.broadcasted_iota(jnp.int32, (D, D), 0)
        ci = lax.broadcasted_iota(jnp.int32, (D, D), 1)
        d = jnp.sum(jnp.where(ri == ci, g, 0.0), axis=1, keepdims=True)

        m2 = m * m
        sum_g2 = jnp.sum(g * g)
        m_g_m = jnp.sum(gm * m)
        sum_m2 = jnp.sum(m2)
        sum_d2 = jnp.sum(d * d)
        sum_dm2 = jnp.sum(d * m2)
        sum_m4 = jnp.sum(m2 * m2)

        off_all = sum_g2 - 2.0 * nf * m_g_m + (nf * sum_m2) * (nf * sum_m2)
        diag_all = sum_d2 - 2.0 * nf * sum_dm2 + nf * nf * sum_m4
        denom = (nf - 1.0) * (nf - 1.0)
        cov_penalty = (off_all - diag_all) / denom / jnp.float32(D)

        counts_c = jnp.maximum(counts, 1.0)
        pos = (counts > 0.0).astype(jnp.float32)
        npos = jnp.sum(pos)
        inv_cc = pos / counts_c
        scalar_loss = jnp.sum(loss_sums * inv_cc) / npos + _COV_W * cov_penalty
        t_ent = jnp.sum(tent_sums * inv_cc) / npos
        kl = jnp.sum((loss_sums - tent_sums) * inv_cc) / npos

        outv = jnp.where(lane8 == 0, scalar_loss,
                         jnp.where(lane8 == 1, t_ent,
                                   jnp.where(lane8 == 2, kl,
                                             jnp.where(lane8 == 3, cov_penalty,
                                                       0.0))))
        out_ref[...] = outv


@jax.jit
def kernel(student_feats, teacher_feats, mask, original_x, center):
    B, D, H, W = student_feats.shape
    HW = H * W
    nb = HW // _PB

    s3 = student_feats.reshape(B, D, HW)
    t3 = teacher_feats.reshape(B, D, HW)
    orig = original_x.reshape(B, 1, HW)
    mask_f = mask.reshape(B, 1, HW).astype(jnp.float32)
    center_col = center.reshape(D, 1)

    out = pl.pallas_call(
        functools.partial(_loss_kernel, nb=nb, b_total=B),
        grid=(B, nb),
        in_specs=[
            pl.BlockSpec((1, D, _PB), lambda b, j: (b, 0, j)),
            pl.BlockSpec((1, D, _PB), lambda b, j: (b, 0, j)),
            pl.BlockSpec((1, 1, _PB), lambda b, j: (b, 0, j)),
            pl.BlockSpec((1, 1, _PB), lambda b, j: (b, 0, j)),
            pl.BlockSpec((D, 1), lambda b, j: (0, 0)),
        ],
        out_specs=pl.BlockSpec((8, 128), lambda b, j: (0, 0)),
        out_shape=jax.ShapeDtypeStruct((8, 128), jnp.float32),
        scratch_shapes=[
            pltpu.VMEM((D, D), jnp.float32),
            pltpu.VMEM((D, 128), jnp.float32),
            pltpu.VMEM((8, 128), jnp.float32),
        ],
        compiler_params=pltpu.CompilerParams(
            dimension_semantics=("arbitrary", "arbitrary")),
    )(s3, t3, orig, mask_f, center_col)

    return (out[0, 0], out[0, 1], out[0, 2], out[0, 3])


# speedup vs baseline: 2.5572x; 1.3755x over previous
import functools

import jax
import jax.numpy as jnp
from jax import lax
from jax.experimental import pallas as pl
from jax.experimental.pallas import tpu as pltpu

_COV_W = 0.001
_PB = 4096


def _loss_kernel(s_ref, t_ref, orig_ref, mask_ref, center_ref, out_ref,
                 gram_acc, sums_acc, img_acc, *, nb, b_total):
    b = pl.program_id(0)
    j = pl.program_id(1)
    D = s_ref.shape[1]

    @pl.when(jnp.logical_and(b == 0, j == 0))
    def _init():
        gram_acc[...] = jnp.zeros_like(gram_acc)
        sums_acc[...] = jnp.zeros_like(sums_acc)
        img_acc[...] = jnp.zeros_like(img_acc)
        out_ref[...] = jnp.zeros_like(out_ref)

    x = s_ref[0]
    y = t_ref[0] - center_ref[...]

    inv_s = 1.0 / jnp.maximum(jnp.sqrt(jnp.sum(x * x, axis=0, keepdims=True)), 1e-12)
    sn = x * inv_s
    inv_t = 1.0 / jnp.maximum(jnp.sqrt(jnp.sum(y * y, axis=0, keepdims=True)), 1e-12)
    tn = y * inv_t

    et = jnp.exp(tn)
    zt = jnp.sum(et, axis=0, keepdims=True)
    log_zt = jnp.log(zt)
    es = jnp.exp(sn)
    log_zs = jnp.log(jnp.sum(es, axis=0, keepdims=True))

    inv_zt = 1.0 / zt
    dot_ts = jnp.sum(et * sn, axis=0, keepdims=True) * inv_zt
    dot_tt = jnp.sum(et * tn, axis=0, keepdims=True) * inv_zt

    fm = jnp.logical_and(orig_ref[0] != 0.0, mask_ref[0] == 0.0)
    fm = fm.astype(jnp.float32)

    loss_px = (log_zs - dot_ts) * fm
    tent_px = (log_zt - dot_tt) * fm

    blk_loss = jnp.sum(loss_px)
    blk_tent = jnp.sum(tent_px)
    blk_cnt = jnp.sum(fm)

    lane8 = lax.broadcasted_iota(jnp.int32, (8, 128), 1)
    sub8 = lax.broadcasted_iota(jnp.int32, (8, 128), 0)
    row = jnp.where(lane8 == 0, blk_loss,
                    jnp.where(lane8 == 1, blk_tent,
                              jnp.where(lane8 == 2, blk_cnt, 0.0)))
    img_acc[...] += jnp.where(sub8 == b, row, 0.0)

    sm = sn * fm
    gram_acc[...] += lax.dot_general(
        sm.astype(jnp.bfloat16), sn.astype(jnp.bfloat16),
        (((1,), (1,)), ((), ())),
        preferred_element_type=jnp.float32)
    laneD = lax.broadcasted_iota(jnp.int32, (D, 128), 1)
    sums_acc[...] += jnp.where(laneD == 0,
                               jnp.sum(sm, axis=1, keepdims=True), 0.0)

    @pl.when(jnp.logical_and(b == b_total - 1, j == nb - 1))
    def _epilogue():
        imgs = img_acc[...]
        loss_sums = imgs[:, 0:1]
        tent_sums = imgs[:, 1:2]
        counts = imgs[:, 2:3]
        nf = jnp.sum(counts)

        g = gram_acc[...]
        m = sums_acc[:, 0:1] / nf
        gm = lax.dot_general(g, m, (((1,), (0,)), ((), ())),
                             preferred_element_type=jnp.float32,
                             precision=lax.Precision.HIGHEST)
        ri = lax.broadcasted_iota(jnp.int32, (D, D), 0)
        ci = lax.broadcasted_iota(jnp.int32, (D, D), 1)
        d = jnp.sum(jnp.where(ri == ci, g, 0.0), axis=1, keepdims=True)

        m2 = m * m
        sum_g2 = jnp.sum(g * g)
        m_g_m = jnp.sum(gm * m)
        sum_m2 = jnp.sum(m2)
        sum_d2 = jnp.sum(d * d)
        sum_dm2 = jnp.sum(d * m2)
        sum_m4 = jnp.sum(m2 * m2)

        off_all = sum_g2 - 2.0 * nf * m_g_m + (nf * sum_m2) * (nf * sum_m2)
        diag_all = sum_d2 - 2.0 * nf * sum_dm2 + nf * nf * sum_m4
        denom = (nf - 1.0) * (nf - 1.0)
        cov_penalty = (off_all - diag_all) / denom / jnp.float32(D)

        counts_c = jnp.maximum(counts, 1.0)
        pos = (counts > 0.0).astype(jnp.float32)
        npos = jnp.sum(pos)
        inv_cc = pos / counts_c
        scalar_loss = jnp.sum(loss_sums * inv_cc) / npos + _COV_W * cov_penalty
        t_ent = jnp.sum(tent_sums * inv_cc) / npos
        kl = jnp.sum((loss_sums - tent_sums) * inv_cc) / npos

        outv = jnp.where(lane8 == 0, scalar_loss,
                         jnp.where(lane8 == 1, t_ent,
                                   jnp.where(lane8 == 2, kl,
                                             jnp.where(lane8 == 3, cov_penalty,
                                                       0.0))))
        out_ref[...] = outv


@jax.jit
def kernel(student_feats, teacher_feats, mask, original_x, center):
    B, D, H, W = student_feats.shape
    HW = H * W
    nb = HW // _PB

    s3 = student_feats.reshape(B, D, HW)
    t3 = teacher_feats.reshape(B, D, HW)
    orig = original_x.reshape(B, 1, HW)
    mask_f = mask.reshape(B, 1, HW).astype(jnp.float32)
    center_col = center.reshape(D, 1)

    out = pl.pallas_call(
        functools.partial(_loss_kernel, nb=nb, b_total=B),
        grid=(B, nb),
        in_specs=[
            pl.BlockSpec((1, D, _PB), lambda b, j: (b, 0, j)),
            pl.BlockSpec((1, D, _PB), lambda b, j: (b, 0, j)),
            pl.BlockSpec((1, 1, _PB), lambda b, j: (b, 0, j)),
            pl.BlockSpec((1, 1, _PB), lambda b, j: (b, 0, j)),
            pl.BlockSpec((D, 1), lambda b, j: (0, 0)),
        ],
        out_specs=pl.BlockSpec((8, 128), lambda b, j: (0, 0)),
        out_shape=jax.ShapeDtypeStruct((8, 128), jnp.float32),
        scratch_shapes=[
            pltpu.VMEM((D, D), jnp.float32),
            pltpu.VMEM((D, 128), jnp.float32),
            pltpu.VMEM((8, 128), jnp.float32),
        ],
        compiler_params=pltpu.CompilerParams(
            dimension_semantics=("arbitrary", "arbitrary")),
    )(s3, t3, orig, mask_f, center_col)

    return (out[0, 0], out[0, 1], out[0, 2], out[0, 3])


# speedup vs baseline: 5.7370x; 2.2435x over previous
import functools

import jax
import jax.numpy as jnp
from jax import lax
from jax.experimental import pallas as pl
from jax.experimental.pallas import tpu as pltpu

_COV_W = 0.001
_HB = 32


def _loss_kernel(s_ref, t_ref, orig_ref, mask_ref, center_ref, out_ref,
                 gram_acc, sums_acc, img_acc, *, nh, b_total):
    b = pl.program_id(0)
    j = pl.program_id(1)
    D = s_ref.shape[1]

    @pl.when(jnp.logical_and(b == 0, j == 0))
    def _init():
        gram_acc[...] = jnp.zeros_like(gram_acc)
        sums_acc[...] = jnp.zeros_like(sums_acc)
        img_acc[...] = jnp.zeros_like(img_acc)
        out_ref[...] = jnp.zeros_like(out_ref)

    x = s_ref[0]
    y = t_ref[0] - center_ref[...]

    inv_s = 1.0 / jnp.maximum(
        jnp.sqrt(jnp.sum(x * x, axis=0, keepdims=True)), 1e-12)
    sn = x * inv_s
    inv_t = 1.0 / jnp.maximum(
        jnp.sqrt(jnp.sum(y * y, axis=0, keepdims=True)), 1e-12)
    tn = y * inv_t

    et = jnp.exp(tn)
    zt = jnp.sum(et, axis=0, keepdims=True)
    log_zt = jnp.log(zt)
    es = jnp.exp(sn)
    log_zs = jnp.log(jnp.sum(es, axis=0, keepdims=True))

    inv_zt = 1.0 / zt
    dot_ts = jnp.sum(et * sn, axis=0, keepdims=True) * inv_zt
    dot_tt = jnp.sum(et * tn, axis=0, keepdims=True) * inv_zt

    fm = jnp.logical_and(orig_ref[0, 0] != 0.0, mask_ref[0] == 0.0)
    fm = fm.astype(jnp.float32)

    loss_r = jnp.sum((log_zs - dot_ts)[0] * fm, axis=0, keepdims=True)
    tent_r = jnp.sum((log_zt - dot_tt)[0] * fm, axis=0, keepdims=True)
    cnt_r = jnp.sum(fm, axis=0, keepdims=True)

    sub24 = lax.broadcasted_iota(jnp.int32, (24, 128), 0)
    img_acc[...] += (jnp.where(sub24 == b, loss_r, 0.0)
                     + jnp.where(sub24 == b + 8, tent_r, 0.0)
                     + jnp.where(sub24 == b + 16, cnt_r, 0.0))

    sm = sn * fm
    sums_acc[...] += jnp.sum(sm, axis=1)
    p = sm.shape[1] * sm.shape[2]
    gram_acc[...] += lax.dot_general(
        sm.astype(jnp.bfloat16).reshape(D, p),
        sn.astype(jnp.bfloat16).reshape(D, p),
        (((1,), (1,)), ((), ())),
        preferred_element_type=jnp.float32)

    @pl.when(jnp.logical_and(b == b_total - 1, j == nh - 1))
    def _epilogue():
        imgs = img_acc[...]
        loss_sums = jnp.sum(imgs[0:8, :], axis=1, keepdims=True)
        tent_sums = jnp.sum(imgs[8:16, :], axis=1, keepdims=True)
        counts = jnp.sum(imgs[16:24, :], axis=1, keepdims=True)
        nf = jnp.sum(counts)

        g = gram_acc[...]
        m = jnp.sum(sums_acc[...], axis=1, keepdims=True) / nf
        gm = lax.dot_general(g, m, (((1,), (0,)), ((), ())),
                             preferred_element_type=jnp.float32,
                             precision=lax.Precision.HIGHEST)
        ri = lax.broadcasted_iota(jnp.int32, (D, D), 0)
        ci = lax.broadcasted_iota(jnp.int32, (D, D), 1)
        d = jnp.sum(jnp.where(ri == ci, g, 0.0), axis=1, keepdims=True)

        m2 = m * m
        sum_g2 = jnp.sum(g * g)
        m_g_m = jnp.sum(gm * m)
        sum_m2 = jnp.sum(m2)
        sum_d2 = jnp.sum(d * d)
        sum_dm2 = jnp.sum(d * m2)
        sum_m4 = jnp.sum(m2 * m2)

        off_all = sum_g2 - 2.0 * nf * m_g_m + (nf * sum_m2) * (nf * sum_m2)
        diag_all = sum_d2 - 2.0 * nf * sum_dm2 + nf * nf * sum_m4
        denom = (nf - 1.0) * (nf - 1.0)
        cov_penalty = (off_all - diag_all) / denom / jnp.float32(D)

        counts_c = jnp.maximum(counts, 1.0)
        pos = (counts > 0.0).astype(jnp.float32)
        npos = jnp.sum(pos)
        inv_cc = pos / counts_c
        scalar_loss = jnp.sum(loss_sums * inv_cc) / npos + _COV_W * cov_penalty
        t_ent = jnp.sum(tent_sums * inv_cc) / npos
        kl = jnp.sum((loss_sums - tent_sums) * inv_cc) / npos

        lane8 = lax.broadcasted_iota(jnp.int32, (8, 128), 1)
        outv = jnp.where(lane8 == 0, scalar_loss,
                         jnp.where(lane8 == 1, t_ent,
                                   jnp.where(lane8 == 2, kl,
                                             jnp.where(lane8 == 3, cov_penalty,
                                                       0.0))))
        out_ref[...] = outv


@jax.jit
def kernel(student_feats, teacher_feats, mask, original_x, center):
    B, D, H, W = student_feats.shape
    nh = H // _HB

    mask_f = mask.astype(jnp.float32)
    center_col = center.reshape(D, 1, 1)

    out = pl.pallas_call(
        functools.partial(_loss_kernel, nh=nh, b_total=B),
        grid=(B, nh),
        in_specs=[
            pl.BlockSpec((1, D, _HB, W), lambda b, j: (b, 0, j, 0)),
            pl.BlockSpec((1, D, _HB, W), lambda b, j: (b, 0, j, 0)),
            pl.BlockSpec((1, 1, _HB, W), lambda b, j: (b, 0, j, 0)),
            pl.BlockSpec((1, _HB, W), lambda b, j: (b, j, 0)),
            pl.BlockSpec((D, 1, 1), lambda b, j: (0, 0, 0)),
        ],
        out_specs=pl.BlockSpec((8, 128), lambda b, j: (0, 0)),
        out_shape=jax.ShapeDtypeStruct((8, 128), jnp.float32),
        scratch_shapes=[
            pltpu.VMEM((D, D), jnp.float32),
            pltpu.VMEM((D, 128), jnp.float32),
            pltpu.VMEM((24, 128), jnp.float32),
        ],
        compiler_params=pltpu.CompilerParams(
            dimension_semantics=("arbitrary", "arbitrary")),
    )(student_feats, teacher_feats, original_x, mask_f, center_col)

    return (out[0, 0], out[0, 1], out[0, 2], out[0, 3])


# speedup vs baseline: 6.6327x; 1.1561x over previous
import functools

import jax
import jax.numpy as jnp
from jax import lax
from jax.experimental import pallas as pl
from jax.experimental.pallas import tpu as pltpu

_COV_W = 0.001
_HB = 32


def _loss_kernel(s_ref, t_ref, orig_ref, mask_ref, center_ref, out_ref,
                 gram_acc, img_acc, *, nh, b_total):
    b = pl.program_id(0)
    j = pl.program_id(1)
    D = s_ref.shape[1]
    HB, W = s_ref.shape[2], s_ref.shape[3]
    P = HB * W

    @pl.when(jnp.logical_and(b == 0, j == 0))
    def _init():
        gram_acc[...] = jnp.zeros_like(gram_acc)
        img_acc[...] = jnp.zeros_like(img_acc)
        out_ref[...] = jnp.zeros_like(out_ref)

    x = s_ref[0]
    y = t_ref[0] - center_ref[...]

    xb = x.astype(jnp.bfloat16).reshape(D, P)
    yb = y.astype(jnp.bfloat16).reshape(D, P)
    ones_row = jnp.ones((1, D), jnp.bfloat16)
    ssx = lax.dot_general(ones_row, xb * xb, (((1,), (0,)), ((), ())),
                          preferred_element_type=jnp.float32)
    ssy = lax.dot_general(ones_row, yb * yb, (((1,), (0,)), ((), ())),
                          preferred_element_type=jnp.float32)

    inv_s2d = 1.0 / jnp.maximum(jnp.sqrt(ssx), 1e-12)
    inv_t2d = 1.0 / jnp.maximum(jnp.sqrt(ssy), 1e-12)

    sn = x * inv_s2d.reshape(1, HB, W)
    tn = y * inv_t2d.reshape(1, HB, W)

    et = jnp.exp(tn)
    zt = jnp.sum(et, axis=0, keepdims=True)
    log_zt = jnp.log(zt)
    es = jnp.exp(sn)
    log_zs = jnp.log(jnp.sum(es, axis=0, keepdims=True))

    inv_zt = 1.0 / zt
    dot_ts = jnp.sum(et * sn, axis=0, keepdims=True) * inv_zt
    dot_tt = jnp.sum(et * tn, axis=0, keepdims=True) * inv_zt

    fm = jnp.logical_and(orig_ref[0, 0] != 0.0, mask_ref[0] == 0.0)
    fm = fm.astype(jnp.float32)

    loss_r = jnp.sum((log_zs - dot_ts)[0] * fm, axis=0, keepdims=True)
    tent_r = jnp.sum((log_zt - dot_tt)[0] * fm, axis=0, keepdims=True)
    cnt_r = jnp.sum(fm, axis=0, keepdims=True)

    sub24 = lax.broadcasted_iota(jnp.int32, (24, 128), 0)
    img_acc[...] += (jnp.where(sub24 == b, loss_r, 0.0)
                     + jnp.where(sub24 == b + 8, tent_r, 0.0)
                     + jnp.where(sub24 == b + 16, cnt_r, 0.0))

    w1 = fm.reshape(1, P) * inv_s2d
    w2b = (w1 * inv_s2d).astype(jnp.bfloat16)
    lhs = xb * w2b
    sub8 = lax.broadcasted_iota(jnp.int32, (8, P), 0)
    pad8 = jnp.where(sub8 == 0, w1, 0.0).astype(jnp.bfloat16)
    lhs_a = jnp.concatenate([lhs, pad8], axis=0)
    gram_acc[...] += lax.dot_general(
        lhs_a, xb, (((1,), (1,)), ((), ())),
        preferred_element_type=jnp.float32)

    @pl.when(jnp.logical_and(b == b_total - 1, j == nh - 1))
    def _epilogue():
        imgs = img_acc[...]
        loss_sums = jnp.sum(imgs[0:8, :], axis=1, keepdims=True)
        tent_sums = jnp.sum(imgs[8:16, :], axis=1, keepdims=True)
        counts = jnp.sum(imgs[16:24, :], axis=1, keepdims=True)
        nf = jnp.sum(counts)

        g = gram_acc[0:D, :]
        m_row = gram_acc[D:D + 1, :] / nf
        gm = lax.dot_general(g, m_row, (((1,), (1,)), ((), ())),
                             preferred_element_type=jnp.float32,
                             precision=lax.Precision.HIGHEST)
        m_g_m = lax.dot_general(m_row, gm, (((1,), (0,)), ((), ())),
                                preferred_element_type=jnp.float32,
                                precision=lax.Precision.HIGHEST)[0, 0]
        ri = lax.broadcasted_iota(jnp.int32, (D, D), 0)
        ci = lax.broadcasted_iota(jnp.int32, (D, D), 1)
        d_row = jnp.sum(jnp.where(ri == ci, g, 0.0), axis=0, keepdims=True)

        m2 = m_row * m_row
        sum_g2 = jnp.sum(g * g)
        sum_m2 = jnp.sum(m2)
        sum_d2 = jnp.sum(d_row * d_row)
        sum_dm2 = jnp.sum(d_row * m2)
        sum_m4 = jnp.sum(m2 * m2)

        off_all = sum_g2 - 2.0 * nf * m_g_m + (nf * sum_m2) * (nf * sum_m2)
        diag_all = sum_d2 - 2.0 * nf * sum_dm2 + nf * nf * sum_m4
        denom = (nf - 1.0) * (nf - 1.0)
        cov_penalty = (off_all - diag_all) / denom / jnp.float32(D)

        counts_c = jnp.maximum(counts, 1.0)
        pos = (counts > 0.0).astype(jnp.float32)
        npos = jnp.sum(pos)
        inv_cc = pos / counts_c
        scalar_loss = jnp.sum(loss_sums * inv_cc) / npos + _COV_W * cov_penalty
        t_ent = jnp.sum(tent_sums * inv_cc) / npos
        kl = jnp.sum((loss_sums - tent_sums) * inv_cc) / npos

        lane8 = lax.broadcasted_iota(jnp.int32, (8, 128), 1)
        outv = jnp.where(lane8 == 0, scalar_loss,
                         jnp.where(lane8 == 1, t_ent,
                                   jnp.where(lane8 == 2, kl,
                                             jnp.where(lane8 == 3, cov_penalty,
                                                       0.0))))
        out_ref[...] = outv


@jax.jit
def kernel(student_feats, teacher_feats, mask, original_x, center):
    B, D, H, W = student_feats.shape
    nh = H // _HB

    mask_f = mask.astype(jnp.float32)
    center_col = center.reshape(D, 1, 1)

    out = pl.pallas_call(
        functools.partial(_loss_kernel, nh=nh, b_total=B),
        grid=(B, nh),
        in_specs=[
            pl.BlockSpec((1, D, _HB, W), lambda b, j: (b, 0, j, 0)),
            pl.BlockSpec((1, D, _HB, W), lambda b, j: (b, 0, j, 0)),
            pl.BlockSpec((1, 1, _HB, W), lambda b, j: (b, 0, j, 0)),
            pl.BlockSpec((1, _HB, W), lambda b, j: (b, j, 0)),
            pl.BlockSpec((D, 1, 1), lambda b, j: (0, 0, 0)),
        ],
        out_specs=pl.BlockSpec((8, 128), lambda b, j: (0, 0)),
        out_shape=jax.ShapeDtypeStruct((8, 128), jnp.float32),
        scratch_shapes=[
            pltpu.VMEM((D + 8, D), jnp.float32),
            pltpu.VMEM((24, 128), jnp.float32),
        ],
        compiler_params=pltpu.CompilerParams(
            dimension_semantics=("arbitrary", "arbitrary")),
    )(student_feats, teacher_feats, original_x, mask_f, center_col)

    return (out[0, 0], out[0, 1], out[0, 2], out[0, 3])
